# baseline (speedup 1.0000x reference)
import functools

import jax
import jax.numpy as jnp
from jax.experimental import pallas as pl
from jax.experimental.pallas import tpu as pltpu

TOPK = 5
LANES = 128
NEG = -1.0e30
BIGI = 2**30


def _body(q_ref, k_ref, ov_ref, oi_ref, qn_ref, rv_ref, ri_ref, *, kb, k_total,
          n_blocks):
    j = pl.program_id(0)
    nq = q_ref.shape[0]

    @pl.when(j == 0)
    def _init():
        rv_ref[:] = jnp.full(rv_ref.shape, NEG, jnp.float32)
        ri_ref[:] = jnp.full(ri_ref.shape, BIGI, jnp.int32)
        q = q_ref[:]
        qn_ref[:] = q / (jnp.sqrt(jnp.sum(q * q, axis=1, keepdims=True)) + 1e-8)

    qn = qn_ref[:]
    k = k_ref[:]
    kn = k / (jnp.sqrt(jnp.sum(k * k, axis=1, keepdims=True)) + 1e-8)
    s = jax.lax.dot_general(
        qn, kn, (((1,), (1,)), ((), ())), preferred_element_type=jnp.float32
    )

    n_sub = kb // LANES
    iota = jax.lax.broadcasted_iota(jnp.int32, (nq, LANES), 1)
    rv = [rv_ref[:, l * LANES:(l + 1) * LANES] for l in range(TOPK)]
    ri = [ri_ref[:, l * LANES:(l + 1) * LANES] for l in range(TOPK)]
    base = j * kb
    for ksub in range(n_sub):
        v = s[:, ksub * LANES:(ksub + 1) * LANES]
        i = iota + (base + ksub * LANES)
        for l in range(TOPK):
            c = v > rv[l]
            nv = jnp.where(c, v, rv[l])
            ni = jnp.where(c, i, ri[l])
            if l < TOPK - 1:
                v = jnp.where(c, rv[l], v)
                i = jnp.where(c, ri[l], i)
            rv[l], ri[l] = nv, ni
    for l in range(TOPK):
        rv_ref[:, l * LANES:(l + 1) * LANES] = rv[l]
        ri_ref[:, l * LANES:(l + 1) * LANES] = ri[l]

    @pl.when(j == n_blocks - 1)
    def _out():
        cv = jnp.concatenate(rv, axis=1)
        ci = jnp.concatenate(ri, axis=1)
        nv, ni = [], []
        for _ in range(TOPK):
            m = jnp.max(cv, axis=1, keepdims=True)
            it = jnp.min(jnp.where(cv == m, ci, BIGI), axis=1, keepdims=True)
            cv = jnp.where(ci == it, NEG, cv)
            nv.append(m)
            ni.append(it)
        pad = ov_ref.shape[1] - TOPK
        ov_ref[:] = jnp.concatenate(
            nv + [jnp.full((nq, pad), NEG, jnp.float32)], axis=1)
        oi_ref[:] = jnp.concatenate(
            ni + [jnp.full((nq, pad), BIGI, jnp.int32)], axis=1)


@jax.jit
def kernel(queries, keys):
    nq, d = queries.shape
    k_total = keys.shape[0]
    kb = 4096
    n_blocks = -(-k_total // kb)
    kp = n_blocks * kb
    if kp != k_total:
        keys = jnp.pad(keys, ((0, kp - k_total), (0, 0)),
                       constant_values=jnp.nan)

    cw = 8
    vals, idx = pl.pallas_call(
        functools.partial(_body, kb=kb, k_total=k_total, n_blocks=n_blocks),
        grid=(n_blocks,),
        in_specs=[
            pl.BlockSpec((nq, d), lambda j: (0, 0)),
            pl.BlockSpec((kb, d), lambda j: (j, 0)),
        ],
        out_specs=[
            pl.BlockSpec((nq, cw), lambda j: (0, 0)),
            pl.BlockSpec((nq, cw), lambda j: (0, 0)),
        ],
        out_shape=[
            jax.ShapeDtypeStruct((nq, cw), jnp.float32),
            jax.ShapeDtypeStruct((nq, cw), jnp.int32),
        ],
        scratch_shapes=[
            pltpu.VMEM((nq, d), jnp.float32),
            pltpu.VMEM((nq, TOPK * LANES), jnp.float32),
            pltpu.VMEM((nq, TOPK * LANES), jnp.int32),
        ],
    )(queries, keys)
    return vals[:, :TOPK], idx[:, :TOPK]


# speedup vs baseline: 1.0123x; 1.0123x over previous
import functools

import jax
import jax.numpy as jnp
from jax.experimental import pallas as pl
from jax.experimental.pallas import tpu as pltpu

TOPK = 5
LANES = 128
NEG = -1.0e30
BIGI = 2**30


def _body(q_ref, k_ref, ov_ref, oi_ref, qn_ref, rv_ref, ri_ref, *, kb, k_total,
          n_blocks):
    j = pl.program_id(0)
    nq = q_ref.shape[0]

    @pl.when(j == 0)
    def _init():
        rv_ref[:] = jnp.full(rv_ref.shape, NEG, jnp.float32)
        ri_ref[:] = jnp.full(ri_ref.shape, BIGI, jnp.int32)
        q = q_ref[:]
        qn_ref[:] = q / (jnp.sqrt(jnp.sum(q * q, axis=1, keepdims=True)) + 1e-8)

    qn = qn_ref[:]
    k = k_ref[:]
    kn = k / (jnp.sqrt(jnp.sum(k * k, axis=1, keepdims=True)) + 1e-8)
    s = jax.lax.dot_general(
        qn, kn, (((1,), (1,)), ((), ())), preferred_element_type=jnp.float32
    )

    n_sub = kb // LANES
    iota = jax.lax.broadcasted_iota(jnp.int32, (nq, LANES), 1)
    rv = [rv_ref[:, l * LANES:(l + 1) * LANES] for l in range(TOPK)]
    ri = [ri_ref[:, l * LANES:(l + 1) * LANES] for l in range(TOPK)]
    base = j * kb
    for ksub in range(n_sub):
        v = s[:, ksub * LANES:(ksub + 1) * LANES]
        i = iota + (base + ksub * LANES)
        for l in range(TOPK):
            c = v > rv[l]
            nv = jnp.where(c, v, rv[l])
            ni = jnp.where(c, i, ri[l])
            if l < TOPK - 1:
                v = jnp.where(c, rv[l], v)
                i = jnp.where(c, ri[l], i)
            rv[l], ri[l] = nv, ni
    for l in range(TOPK):
        rv_ref[:, l * LANES:(l + 1) * LANES] = rv[l]
        ri_ref[:, l * LANES:(l + 1) * LANES] = ri[l]

    @pl.when(j == n_blocks - 1)
    def _out():
        cv = jnp.concatenate(rv, axis=1)
        ci = jnp.concatenate(ri, axis=1)
        nv, ni = [], []
        for _ in range(TOPK):
            m = jnp.max(cv, axis=1, keepdims=True)
            it = jnp.min(jnp.where(cv == m, ci, BIGI), axis=1, keepdims=True)
            cv = jnp.where(ci == it, NEG, cv)
            nv.append(m)
            ni.append(it)
        pad = ov_ref.shape[1] - TOPK
        ov_ref[:] = jnp.concatenate(
            nv + [jnp.full((nq, pad), NEG, jnp.float32)], axis=1)
        oi_ref[:] = jnp.concatenate(
            ni + [jnp.full((nq, pad), BIGI, jnp.int32)], axis=1)


@jax.jit
def kernel(queries, keys):
    nq, d = queries.shape
    k_total = keys.shape[0]
    kb = 2048
    n_blocks = -(-k_total // kb)
    kp = n_blocks * kb
    if kp != k_total:
        keys = jnp.pad(keys, ((0, kp - k_total), (0, 0)),
                       constant_values=jnp.nan)

    cw = 8
    vals, idx = pl.pallas_call(
        functools.partial(_body, kb=kb, k_total=k_total, n_blocks=n_blocks),
        grid=(n_blocks,),
        in_specs=[
            pl.BlockSpec((nq, d), lambda j: (0, 0)),
            pl.BlockSpec((kb, d), lambda j: (j, 0)),
        ],
        out_specs=[
            pl.BlockSpec((nq, cw), lambda j: (0, 0)),
            pl.BlockSpec((nq, cw), lambda j: (0, 0)),
        ],
        out_shape=[
            jax.ShapeDtypeStruct((nq, cw), jnp.float32),
            jax.ShapeDtypeStruct((nq, cw), jnp.int32),
        ],
        scratch_shapes=[
            pltpu.VMEM((nq, d), jnp.float32),
            pltpu.VMEM((nq, TOPK * LANES), jnp.float32),
            pltpu.VMEM((nq, TOPK * LANES), jnp.int32),
        ],
    )(queries, keys)
    return vals[:, :TOPK], idx[:, :TOPK]
